# baseline (speedup 1.0000x reference)
import jax
import jax.numpy as jnp
from jax import lax
from jax.experimental import pallas as pl
from jax.experimental.pallas import tpu as pltpu
from jax.experimental.pallas import tpu_sc as plsc

_VOCAB = 32000
_EXTRA = 512
_ROW = _VOCAB + _EXTRA
_N = 4096
_EPS = 1e-20
_LN2 = 0.6931471805599453
_SQRT2 = 1.4142135623730951

_NC, _NS, _L = 1, 16, 16
_TILES_PER_ROW = _ROW // 128
_NW = _NC * _NS
_RPW = _N // _NW
_CH = _RPW // _L
_G = 128
_NB = _RPW // _G

_OFF_MASK = (1 << 27) - 1
_B_UNK = 1 << 30
_B_NZ = 1 << 29
_B_IGN = 1 << 28


def _neg_log(p):
    bits = lax.bitcast_convert_type(p, jnp.int32)
    e = (bits >> 23) - 127
    m = lax.bitcast_convert_type((bits & 0x007FFFFF) | 0x3F800000, jnp.float32)
    ef = e.astype(jnp.float32)
    big = m >= _SQRT2
    m = jnp.where(big, m * 0.5, m)
    ef = jnp.where(big, ef + 1.0, ef)
    t = m - 1.0
    s = t / (2.0 + t)
    z = s * s
    poly = (1.0 / 3.0) + z * (0.2 + z * ((1.0 / 7.0) + z * (1.0 / 9.0)))
    return -(ef * _LN2 + (2.0 * s) * (1.0 + z * poly))


def _body(scores_hbm, ti_hbm, ci_hbm, out_hbm,
          ti_v, cf_v, ci_v, vg_v, cg_v, ls_v, s_a, s_b, s_c, s_d):
    wid = lax.axis_index("s") * _NC + lax.axis_index("c")
    base = wid * _NB
    c1 = pltpu.async_copy(ti_hbm.at[pl.ds(base, _NB)], ti_v, s_a)
    c2 = pltpu.async_copy(ci_hbm.at[pl.ds(base, _NB)], cf_v, s_b)
    c1.wait()
    gt = [pltpu.async_copy(scores_hbm.at[ti_v.at[k]], vg_v.at[k], s)
          for k, s in ((0, s_a), (1, s_c))]
    c2.wait()
    gc = []
    for k, s in ((0, s_b), (1, s_d)):
        for jj in range(_G // _L):
            o = jj * _L
            ci_v[k, pl.ds(o, _L)] = cf_v[k, pl.ds(o, _L)] & _OFF_MASK
        gc.append(pltpu.async_copy(scores_hbm.at[ci_v.at[k]], cg_v.at[k], s))
    outs = []
    for k in range(_NB):
        gt[k].wait()
        gc[k].wait()
        for jj in range(_G // _L):
            oo = jj * _L
            o = k * _G + oo
            fl = cf_v[k, pl.ds(oo, _L)]
            vp = vg_v[k, pl.ds(oo, _L)]
            cp = cg_v[k, pl.ds(oo, _L)]
            a_unk = (fl & _B_UNK) != 0
            cpe = jnp.where(a_unk, 0.0, cp) + _EPS
            non_copy = (fl & (_B_UNK | _B_NZ)) != 0
            p = jnp.where(non_copy, cpe + vp, cpe)
            loss = _neg_log(p)
            ls_v[pl.ds(o, _L)] = jnp.where((fl & _B_IGN) != 0, 0.0, loss)
        outs.append(pltpu.async_copy(
            ls_v.at[pl.ds(k * _G, _G)],
            out_hbm.at[pl.ds(wid * _RPW + k * _G, _G)],
            (s_a, s_c)[k]))
    for w in outs:
        w.wait()


_sc_loss = pl.kernel(
    _body,
    mesh=plsc.VectorSubcoreMesh(core_axis_name="c", subcore_axis_name="s",
                                num_cores=1),
    out_type=jax.ShapeDtypeStruct((_N,), jnp.float32),
    scratch_types=[
        pltpu.VMEM((_NB, _G), jnp.int32),
        pltpu.VMEM((_NB, _G), jnp.int32),
        pltpu.VMEM((_NB, _G), jnp.int32),
        pltpu.VMEM((_NB, _G), jnp.float32),
        pltpu.VMEM((_NB, _G), jnp.float32),
        pltpu.VMEM((_RPW,), jnp.float32),
        pltpu.SemaphoreType.DMA,
        pltpu.SemaphoreType.DMA,
        pltpu.SemaphoreType.DMA,
        pltpu.SemaphoreType.DMA,
    ],
)


def kernel(scores, align, target):
    flat = (scores.reshape(_N // 8, 8, _TILES_PER_ROW, 128)
            .transpose(0, 2, 1, 3).reshape(-1))
    align = align.astype(jnp.int32)
    target = target.astype(jnp.int32)
    rows = jnp.arange(_N, dtype=jnp.int32)
    rbase = (rows >> 3) * (_TILES_PER_ROW * 1024) + ((rows & 7) << 7)
    t0 = jnp.maximum(target, 0)
    ti = rbase + ((t0 >> 7) << 10) + (t0 & 127)
    c0 = _VOCAB + align
    flags = (jnp.where(align == 0, _B_UNK, 0)
             | jnp.where(target != 0, _B_NZ, 0)
             | jnp.where(target == -100, _B_IGN, 0))
    ci = (rbase + ((c0 >> 7) << 10) + (c0 & 127)) | flags
    return _sc_loss(flat, ti.reshape(-1, _G), ci.reshape(-1, _G))


# speedup vs baseline: 1.0085x; 1.0085x over previous
import jax
import jax.numpy as jnp
from jax import lax
from jax.experimental import pallas as pl
from jax.experimental.pallas import tpu as pltpu
from jax.experimental.pallas import tpu_sc as plsc

_VOCAB = 32000
_EXTRA = 512
_ROW = _VOCAB + _EXTRA
_N = 4096
_EPS = 1e-20
_LN2 = 0.6931471805599453
_SQRT2 = 1.4142135623730951

_NC, _NS, _L = 1, 16, 16
_TILES_PER_ROW = _ROW // 128
_NW = _NC * _NS
_RPW = _N // _NW
_CH = _RPW // _L
_G = 128
_NB = _RPW // _G

_OFF_MASK = (1 << 27) - 1
_B_UNK = 1 << 30
_B_NZ = 1 << 29
_B_IGN = 1 << 28


def _neg_log(p):
    bits = lax.bitcast_convert_type(p, jnp.int32)
    e = (bits >> 23) - 127
    m = lax.bitcast_convert_type((bits & 0x007FFFFF) | 0x3F800000, jnp.float32)
    ef = e.astype(jnp.float32)
    big = m >= _SQRT2
    m = jnp.where(big, m * 0.5, m)
    ef = jnp.where(big, ef + 1.0, ef)
    t = m - 1.0
    s = t / (2.0 + t)
    z = s * s
    poly = (1.0 / 3.0) + z * (0.2 + z * ((1.0 / 7.0) + z * (1.0 / 9.0)))
    return -(ef * _LN2 + (2.0 * s) * (1.0 + z * poly))


def _body(scores_hbm, ti_hbm, ci_hbm, out_hbm,
          ti_v, cf_v, ci_v, vg_v, cg_v, ls_v, sem_t, sem_c):
    wid = lax.axis_index("s") * _NC + lax.axis_index("c")
    base = wid * _NB
    with jax.named_scope("stage"):
        c1 = pltpu.async_copy(ti_hbm.at[pl.ds(base, _NB)], ti_v, sem_t)
        c2 = pltpu.async_copy(ci_hbm.at[pl.ds(base, _NB)], cf_v, sem_c)
        c1.wait()
    with jax.named_scope("fire_t"):
        waits = [pltpu.async_copy(scores_hbm.at[ti_v.at[k]], vg_v.at[k], sem_t)
                 for k in range(_NB)]
        c2.wait()
    with jax.named_scope("strip_fire_c"):
        for k in range(_NB):
            for jj in range(_G // _L):
                o = jj * _L
                ci_v[k, pl.ds(o, _L)] = cf_v[k, pl.ds(o, _L)] & _OFF_MASK
            waits.append(pltpu.async_copy(scores_hbm.at[ci_v.at[k]],
                                          cg_v.at[k], sem_c))
    with jax.named_scope("gather_wait"):
        for w in waits:
            w.wait()
    for j in range(_CH):
        o = j * _L
        k, oo = j // (_G // _L), (j % (_G // _L)) * _L
        fl = cf_v[k, pl.ds(oo, _L)]
        vp = vg_v[k, pl.ds(oo, _L)]
        cp = cg_v[k, pl.ds(oo, _L)]
        a_unk = (fl & _B_UNK) != 0
        cpe = jnp.where(a_unk, 0.0, cp) + _EPS
        non_copy = (fl & (_B_UNK | _B_NZ)) != 0
        p = jnp.where(non_copy, cpe + vp, cpe)
        loss = _neg_log(p)
        ls_v[pl.ds(o, _L)] = jnp.where((fl & _B_IGN) != 0, 0.0, loss)
    with jax.named_scope("out"):
        pltpu.sync_copy(ls_v, out_hbm.at[pl.ds(wid * _RPW, _RPW)])


_sc_loss = pl.kernel(
    _body,
    mesh=plsc.VectorSubcoreMesh(core_axis_name="c", subcore_axis_name="s",
                                num_cores=1),
    out_type=jax.ShapeDtypeStruct((_N,), jnp.float32),
    scratch_types=[
        pltpu.VMEM((_NB, _G), jnp.int32),
        pltpu.VMEM((_NB, _G), jnp.int32),
        pltpu.VMEM((_NB, _G), jnp.int32),
        pltpu.VMEM((_NB, _G), jnp.float32),
        pltpu.VMEM((_NB, _G), jnp.float32),
        pltpu.VMEM((_RPW,), jnp.float32),
        pltpu.SemaphoreType.DMA,
        pltpu.SemaphoreType.DMA,
    ],
)


def kernel(scores, align, target):
    flat = (scores.reshape(_N // 8, 8, _TILES_PER_ROW, 128)
            .transpose(0, 2, 1, 3).reshape(-1))
    align = align.astype(jnp.int32)
    target = target.astype(jnp.int32)
    rows = jnp.arange(_N, dtype=jnp.int32)
    rbase = (rows >> 3) * (_TILES_PER_ROW * 1024) + ((rows & 7) << 7)
    t0 = jnp.maximum(target, 0)
    ti = rbase + ((t0 >> 7) << 10) + (t0 & 127)
    c0 = _VOCAB + align
    flags = (jnp.where(align == 0, _B_UNK, 0)
             | jnp.where(target != 0, _B_NZ, 0)
             | jnp.where(target == -100, _B_IGN, 0))
    ci = (rbase + ((c0 >> 7) << 10) + (c0 & 127)) | flags
    return _sc_loss(flat, ti.reshape(-1, _G), ci.reshape(-1, _G))
